# Initial kernel scaffold; baseline (speedup 1.0000x reference)
import functools

import jax
import jax.numpy as jnp
from jax import lax
from jax.experimental import pallas as pl
from jax.experimental.pallas import tpu as pltpu
from jax.experimental.pallas import tpu_sc as plsc

D = 64
B = 1024
L_I = 50
L_S = 20
L_IU = 50
NW = 32

F32 = jnp.float32


_U_TASKS = ((B, 0), (B * L_S, B), (B * L_IU, B + B * L_S))
_I_TASKS = ((B, 0), (B * L_I, B), (B * 400, B + B * L_I))
_CHUNK = 512


def _task_chunk(nt):
    c = min(nt, _CHUNK)
    while nt % c:
        c -= 8
    return c


def _sc_gather(embed_u, embed_i, idx_u, idx_i):
    nu = idx_u.shape[0]
    ni = idx_i.shape[0]
    out_type = [jax.ShapeDtypeStruct((n, D), F32) for n, _ in _U_TASKS] + \
               [jax.ShapeDtypeStruct((n, D), F32) for n, _ in _I_TASKS]
    mesh = plsc.VectorSubcoreMesh(core_axis_name="c", subcore_axis_name="s")

    @functools.partial(
        pl.kernel, mesh=mesh, out_type=out_type,
        scratch_types=[
            pltpu.VMEM((max(nu, ni) // NW,), jnp.int32),
            pltpu.VMEM((_CHUNK, D), F32),
            pltpu.SemaphoreType.DMA,
        ],
    )
    def k(eu, ei, iu, ii, *rest):
        outs = rest[:6]
        idxbuf, rowbuf, sem = rest[6:]
        wid = lax.axis_index("s") * 2 + lax.axis_index("c")

        def run_task(table, idx_hbm, idx_base, out, n):
            nt = n // NW
            base = wid * nt
            pltpu.sync_copy(idx_hbm.at[pl.ds(idx_base + base, nt)],
                            idxbuf.at[pl.ds(0, nt)])
            c = _task_chunk(nt)
            nchunks = nt // c

            def body(j, carry):
                off = j * c
                pltpu.async_copy(
                    table.at[idxbuf.at[pl.ds(off, c)]],
                    rowbuf.at[pl.ds(0, c)], sem).wait()
                pltpu.sync_copy(rowbuf.at[pl.ds(0, c)],
                                out.at[pl.ds(base + off, c)])
                return carry

            if nchunks == 1:
                body(0, 0)
            else:
                lax.fori_loop(0, nchunks, body, 0)

        for t, (n, ib) in enumerate(_U_TASKS):
            run_task(eu, iu, ib, outs[t], n)
        for t, (n, ib) in enumerate(_I_TASKS):
            run_task(ei, ii, ib, outs[3 + t], n)

    return k(embed_u, embed_i, idx_u, idx_i)


_BB1 = 32


def _pass1_stream(rows_ref, r_ref, self_ref, w_refs, x_out_ref, l_out_ref,
                  L, M, bb):
    W1a, rtab, W2, b2, Wf, bf, A1t, c1, a2, c2 = (w[...] for w in w_refs)
    rows = rows_ref[...]
    r = r_ref[...].reshape(bb * L, 1)
    oh = (r == lax.broadcasted_iota(jnp.int32, (1, 8), 1)).astype(F32)
    x1 = jnp.maximum(
        jnp.dot(rows, W1a, preferred_element_type=F32)
        + jnp.dot(oh, rtab, preferred_element_type=F32), 0.0)
    x2 = jnp.maximum(jnp.dot(x1, W2, preferred_element_type=F32) + b2, 0.0)
    y = jnp.dot(x2, Wf, preferred_element_type=F32) + bf
    x3 = y[:, :D]
    hn = y[:, D:]
    selfr = self_ref[...]
    stop = jnp.dot(selfr, A1t, preferred_element_type=F32) + c1
    stop_f = jnp.broadcast_to(
        stop.reshape(bb, 1, M, D), (bb, L // M, M, D)).reshape(bb * L, D)
    h4 = jnp.maximum(hn + stop_f, 0.0) * a2
    logit = jnp.sum(h4.reshape(bb, L, D), axis=-1) + c2
    x_out_ref[...] = x3
    l_out_ref[...] = logit


def _pass1_body(qa, ria, pi, qao, ro, po, pt, rit, qj, *rest):
    wI = rest[0:10]
    wO = rest[10:20]
    wit = rest[20:30]
    xia, lI, xoa, lO, fjt, lit = rest[30:36]
    _pass1_stream(qa, ria, pi, wI, xia, lI, L_I, 1, _BB1)
    _pass1_stream(qao, ro, po, wO, xoa, lO, 400, L_S, _BB1)
    _pass1_stream(pt, rit, qj, wit, fjt, lit, L_IU, 1, _BB1)


def _pass1(qa, ria, pi, qao, ro, po, pt, rit, qj, wI, wO, wit):
    nb = B // _BB1
    bb = _BB1

    def rows_spec(L):
        return pl.BlockSpec((bb * L, D), lambda i: (i, 0))

    def ridx_spec(L):
        return pl.BlockSpec((bb, L), lambda i: (i, 0))

    def w_specs(ws):
        return [pl.BlockSpec(w.shape, lambda i: (0, 0)) for w in ws]

    in_specs = (
        [rows_spec(L_I), ridx_spec(L_I), rows_spec(1),
         rows_spec(400), ridx_spec(400), rows_spec(L_S),
         rows_spec(L_IU), ridx_spec(L_IU), rows_spec(1)]
        + w_specs(wI) + w_specs(wO) + w_specs(wit))
    out_specs = [rows_spec(L_I), ridx_spec(L_I),
                 rows_spec(400), ridx_spec(400),
                 rows_spec(L_IU), ridx_spec(L_IU)]
    out_shape = [
        jax.ShapeDtypeStruct((B * L_I, D), F32),
        jax.ShapeDtypeStruct((B, L_I), F32),
        jax.ShapeDtypeStruct((B * 400, D), F32),
        jax.ShapeDtypeStruct((B, 400), F32),
        jax.ShapeDtypeStruct((B * L_IU, D), F32),
        jax.ShapeDtypeStruct((B, L_IU), F32),
    ]
    return pl.pallas_call(
        _pass1_body,
        grid=(nb,),
        in_specs=in_specs,
        out_specs=out_specs,
        out_shape=out_shape,
    )(qa, ria, pi, qao, ro, po, pt, rit, qj, *wI, *wO, *wit)


def _stats_body(lI, lO, lit, sI, sO, sit):
    for l_ref, s_ref in ((lI, sI), (lO, sO), (lit, sit)):
        l = l_ref[...]
        m = jnp.max(l, axis=0, keepdims=True)
        s = jnp.sum(jnp.exp(l - m), axis=0, keepdims=True)
        s_ref[...] = jnp.concatenate([m, s], axis=0)


def _stats(lI, lO, lit):
    return pl.pallas_call(
        _stats_body,
        out_shape=[jax.ShapeDtypeStruct((2, L_I), F32),
                   jax.ShapeDtypeStruct((2, 400), F32),
                   jax.ShapeDtypeStruct((2, L_IU), F32)],
    )(lI, lO, lit)


_BB2 = 64


def _agg(x_ref, l_ref, st_ref, W, b, L, M, bb):
    st = st_ref[...]
    a = jnp.exp(l_ref[...] - st[0:1]) / st[1:2]
    x = x_ref[...].reshape(bb, L // M, M, D)
    S = jnp.sum(a.reshape(bb, L // M, M, 1) * x, axis=1)
    return jnp.maximum(
        jnp.dot(S.reshape(bb * M, D), W, preferred_element_type=F32) + b, 0.0)


def _pass2_body(xia, lI, xoa, lO, fjt, lit, sI, sO, sit, pi,
                WI, bI, WO, bO, WL, bL, A1St, A1Sb, c1S, a2S, c2S,
                hiI_o, hoI_o, hb_o, zj_o):
    bb = _BB2
    hiI_o[...] = _agg(xia, lI, sI, WI[...], bI[...], L_I, 1, bb)
    hoI = _agg(xoa, lO, sO, WO[...], bO[...], 400, L_S, bb)
    hoI_o[...] = hoI
    zj_o[...] = _agg(fjt, lit, sit, WL[...], bL[...], L_IU, 1, bb)
    piA = jnp.dot(pi[...], A1St[...], preferred_element_type=F32) + c1S[...]
    hob = jnp.dot(hoI, A1Sb[...], preferred_element_type=F32)
    h = jnp.maximum(hob.reshape(bb, L_S, D) + piA.reshape(bb, 1, D), 0.0)
    hb_o[...] = jnp.sum(h * a2S[...], axis=-1) + c2S[0, 0]


def _pass2(xia, lI, xoa, lO, fjt, lit, sI, sO, sit, pi, ws):
    bb = _BB2
    nb = B // bb

    def rows_spec(L):
        return pl.BlockSpec((bb * L, D), lambda i: (i, 0))

    def l_spec(L):
        return pl.BlockSpec((bb, L), lambda i: (i, 0))

    def full(a):
        return pl.BlockSpec(a.shape, lambda i: (0, 0))

    in_specs = ([rows_spec(L_I), l_spec(L_I), rows_spec(400), l_spec(400),
                 rows_spec(L_IU), l_spec(L_IU)]
                + [full(s) for s in (sI, sO, sit)] + [rows_spec(1)]
                + [full(w) for w in ws])
    out_specs = [rows_spec(1), rows_spec(L_S), l_spec(L_S), rows_spec(1)]
    out_shape = [jax.ShapeDtypeStruct((B, D), F32),
                 jax.ShapeDtypeStruct((B * L_S, D), F32),
                 jax.ShapeDtypeStruct((B, L_S), F32),
                 jax.ShapeDtypeStruct((B, D), F32)]
    return pl.pallas_call(
        _pass2_body,
        grid=(nb,),
        in_specs=in_specs,
        out_specs=out_specs,
        out_shape=out_shape,
    )(xia, lI, xoa, lO, fjt, lit, sI, sO, sit, pi, *ws)


def _mlp3(x, W1, b1, W2, b2, W3, b3):
    h = jnp.maximum(jnp.dot(x, W1, preferred_element_type=F32) + b1, 0.0)
    h = jnp.maximum(jnp.dot(h, W2, preferred_element_type=F32) + b2, 0.0)
    return jnp.dot(h, W3, preferred_element_type=F32) + b3


def _pass3_body(hiI, hoI, hb, zj,
                WO, bO, W1u, b1u, W2u, b2u, W3u, b3u,
                w1g, b1g, w2g, b2g, w3g, b3g, out):
    h = hb[...]
    m = jnp.max(h, axis=0, keepdims=True)
    e = jnp.exp(h - m)
    beta = e / jnp.sum(e, axis=0, keepdims=True)
    ho = hoI[...].reshape(B, L_S, D)
    Sp = jnp.sum(beta.reshape(B, L_S, 1) * ho, axis=1)
    hiS = jnp.maximum(
        jnp.dot(Sp, WO[...], preferred_element_type=F32) + bO[...], 0.0)
    xu = jnp.concatenate([hiI[...], hiS], axis=1)
    hi = _mlp3(xu, W1u[...], b1u[...], W2u[...], b2u[...], W3u[...], b3u[...])
    xg = jnp.concatenate([hi, zj[...]], axis=1)
    g1 = jnp.maximum(
        jnp.sum(xg * w1g[...], axis=1, keepdims=True) + b1g[0, 0], 0.0)
    g2 = jnp.maximum(g1 * w2g[0, 0] + b2g[0, 0], 0.0)
    out[...] = g2 * w3g[0, 0] + b3g[0, 0]


def _pass3(hiI, hoI, hb, zj, ws):
    return pl.pallas_call(
        _pass3_body,
        out_shape=jax.ShapeDtypeStruct((B, 1), F32),
    )(hiI, hoI, hb, zj, *ws)


def _row2(v):
    return v.reshape(1, -1)


def _prep_stream(embed_r, mlp, att):
    W1, b1, W2, b2, W3, b3 = mlp
    A1, c1, A2, c2 = att
    rtab = embed_r @ W1[D:] + b1
    rtab8 = jnp.concatenate([rtab, jnp.zeros((3, D), F32)], axis=0)
    Wf = jnp.concatenate([W3, W3 @ A1[D:]], axis=1)
    bf = jnp.concatenate([b3, b3 @ A1[D:]], axis=0)
    return (W1[:D], rtab8, W2, _row2(b2), Wf, _row2(bf),
            A1[:D], _row2(c1), _row2(A2[:, 0]), _row2(c2))


def kernel(params, nodes_u, nodes_i, u_items_list, u_users_list, i_users_list,
           u_items_r, u_users_items, u_users_items_r, i_users_r):
    p = params
    i32 = jnp.int32
    idx_u = jnp.concatenate([nodes_u.astype(i32),
                             u_users_list.astype(i32).reshape(-1),
                             i_users_list.astype(i32).reshape(-1)])
    idx_i = jnp.concatenate([nodes_i.astype(i32),
                             u_items_list.astype(i32).reshape(-1),
                             u_users_items.astype(i32).reshape(-1)])

    pi, po, pt, qj, qa, qao = _sc_gather(p['embed_u'], p['embed_i'],
                                         idx_u, idx_i)

    wI = _prep_stream(p['embed_r'], p['gv'], p['att_I'])
    wO = _prep_stream(p['embed_r'], p['gv'], p['att_O'])
    wit = _prep_stream(p['embed_r'], p['gu'], p['att_item'])

    ria = u_items_r.astype(i32)
    ro = u_users_items_r.astype(i32).reshape(B, 400)
    rit = i_users_r.astype(i32)

    xia, lI, xoa, lO, fjt, lit = _pass1(qa, ria, pi, qao, ro, po,
                                        pt, rit, qj, wI, wO, wit)
    sI, sO, sit = _stats(lI, lO, lit)

    WI, bI = p['linI']
    WO, bO = p['linO']
    WL, bL = p['lin_item']
    A1S, c1S, A2S, c2S = p['att_S']
    ws2 = [WI, _row2(bI), WO, _row2(bO), WL, _row2(bL),
           A1S[:D], A1S[D:], _row2(c1S), _row2(A2S[:, 0]),
           c2S.reshape(1, 1)]
    hiI, hoI, hb, zj = _pass2(xia, lI, xoa, lO, fjt, lit, sI, sO, sit, pi, ws2)

    W1u, b1u, W2u, b2u, W3u, b3u = p['umlp']
    w1g, b1g, w2g, b2g, w3g, b3g = p['g']
    ws3 = [WO, _row2(bO), W1u, _row2(b1u), W2u, _row2(b2u), W3u, _row2(b3u),
           _row2(w1g[:, 0]), b1g.reshape(1, 1), w2g.reshape(1, 1),
           b2g.reshape(1, 1), w3g.reshape(1, 1), b3g.reshape(1, 1)]
    return _pass3(hiI, hoI, hb, zj, ws3)


# speedup vs baseline: 3.8522x; 3.8522x over previous
import functools

import jax
import jax.numpy as jnp
from jax import lax
from jax.experimental import pallas as pl
from jax.experimental.pallas import tpu as pltpu
from jax.experimental.pallas import tpu_sc as plsc

D = 64
B = 1024
L_I = 50
L_S = 20
L_IU = 50
NW = 32
NU = 100000

F32 = jnp.float32
BF16 = jnp.bfloat16


_PREP_BLK = 2000


def _prep_body(eu, ei, wu, wi, tu, ti):
    for e_ref, w_ref, t_ref in ((eu, wu, tu), (ei, wi, ti)):
        e = e_ref[...]
        t = jnp.dot(e, w_ref[...], preferred_element_type=F32)
        t_ref[...] = jnp.concatenate([t, e], axis=1)


def _prep_tables(embed_u, embed_i, w1a_gu, w1a_gv):
    blk = pl.BlockSpec((_PREP_BLK, D), lambda i: (i, 0))
    wblk = pl.BlockSpec((D, D), lambda i: (0, 0))
    oblk = pl.BlockSpec((_PREP_BLK, 2 * D), lambda i: (i, 0))
    return pl.pallas_call(
        _prep_body,
        grid=(NU // _PREP_BLK,),
        in_specs=[blk, blk, wblk, wblk],
        out_specs=[oblk, oblk],
        out_shape=[jax.ShapeDtypeStruct((NU, 2 * D), F32),
                   jax.ShapeDtypeStruct((NU, 2 * D), F32)],
    )(embed_u, embed_i, w1a_gu, w1a_gv)


_U_TASKS = ((B, 0), (B * L_S, B), (B * L_IU, B + B * L_S))
_I_TASKS = ((B, 0), (B * L_I, B), (B * 400, B + B * L_I))
_CHUNK = 512


def _task_chunk(nt):
    c = min(nt, _CHUNK)
    while nt % c:
        c -= 8
    return c


def _sc_gather(tbl_u, tbl_i, idx_u, idx_i):
    nu = idx_u.shape[0]
    ni = idx_i.shape[0]
    out_type = [jax.ShapeDtypeStruct((n, 2 * D), F32) for n, _ in _U_TASKS] + \
               [jax.ShapeDtypeStruct((n, 2 * D), F32) for n, _ in _I_TASKS]
    mesh = plsc.VectorSubcoreMesh(core_axis_name="c", subcore_axis_name="s")

    @functools.partial(
        pl.kernel, mesh=mesh, out_type=out_type,
        scratch_types=[
            pltpu.VMEM((max(nu, ni) // NW,), jnp.int32),
            pltpu.VMEM((_CHUNK, 2 * D), F32),
            pltpu.SemaphoreType.DMA,
        ],
    )
    def k(tu, ti, iu, ii, *rest):
        outs = rest[:6]
        idxbuf, rowbuf, sem = rest[6:]
        wid = lax.axis_index("s") * 2 + lax.axis_index("c")

        def run_task(table, idx_hbm, idx_base, out, n):
            nt = n // NW
            base = wid * nt
            pltpu.sync_copy(idx_hbm.at[pl.ds(idx_base + base, nt)],
                            idxbuf.at[pl.ds(0, nt)])
            c = _task_chunk(nt)
            nchunks = nt // c

            def body(j, carry):
                off = j * c
                pltpu.async_copy(
                    table.at[idxbuf.at[pl.ds(off, c)]],
                    rowbuf.at[pl.ds(0, c)], sem).wait()
                pltpu.sync_copy(rowbuf.at[pl.ds(0, c)],
                                out.at[pl.ds(base + off, c)])
                return carry

            if nchunks == 1:
                body(0, 0)
            else:
                lax.fori_loop(0, nchunks, body, 0)

        for t, (n, ib) in enumerate(_U_TASKS):
            run_task(tu, iu, ib, outs[t], n)
        for t, (n, ib) in enumerate(_I_TASKS):
            run_task(ti, ii, ib, outs[3 + t], n)

    return k(tbl_u, tbl_i, idx_u, idx_i)


_BB1 = 32


def _pack_pairs(x, n):
    x3 = x.reshape(n // 2, 2, D)
    return jnp.concatenate([x3[:, 0, :], x3[:, 1, :]], axis=1)


def _unpack_pairs(x2, n):
    a = x2[:, :D].reshape(n // 2, 1, D)
    b = x2[:, D:].reshape(n // 2, 1, D)
    return jnp.concatenate([a, b], axis=1).reshape(n, D)


def _pass1_stream(rows_ref, r_ref, self_ref, w_refs, x_out_ref, l_out_ref,
                  L, M, bb):
    rtab, W2, b2, Wf, bf, A1t, c1, a2, c2 = (w[...] for w in w_refs)
    n = bb * L
    t = rows_ref[...][:, :D]
    r3 = lax.broadcast_in_dim(r_ref[...], (bb, L, 8), (0, 1))
    oh = (r3 == lax.broadcasted_iota(jnp.int32, (bb, L, 8), 2)
          ).astype(F32).reshape(n, 8)
    x1 = jnp.maximum(t + jnp.dot(oh, rtab, preferred_element_type=F32), 0.0)
    x2 = jnp.maximum(jnp.dot(x1, W2, preferred_element_type=F32) + b2, 0.0)
    y = jnp.dot(x2, Wf, preferred_element_type=F32) + bf
    x3 = y[:, :D]
    hn = y[:, D:]
    selfr = self_ref[...][:, D:]
    stop = jnp.dot(selfr, A1t, preferred_element_type=F32) + c1
    stop_f = jnp.broadcast_to(
        stop.reshape(bb, 1, M, D), (bb, L // M, M, D)).reshape(n, D)
    h4 = jnp.maximum(hn + stop_f, 0.0) * a2
    logit = jnp.sum(h4.reshape(bb, L, D), axis=-1) + c2
    x_out_ref[...] = _pack_pairs(x3, n).astype(BF16)
    l_out_ref[...] = logit


def _pass1_body(qa, ria, pi, qao, ro, po, pt, rit, qj, *rest):
    wI = rest[0:9]
    wO = rest[9:18]
    wit = rest[18:27]
    xia, lI, xoa, lO, fjt, lit = rest[27:33]
    _pass1_stream(qa, ria, pi, wI, xia, lI, L_I, 1, _BB1)
    _pass1_stream(qao, ro, po, wO, xoa, lO, 400, L_S, _BB1)
    _pass1_stream(pt, rit, qj, wit, fjt, lit, L_IU, 1, _BB1)


def _pass1(qa, ria, pi, qao, ro, po, pt, rit, qj, wI, wO, wit):
    nb = B // _BB1
    bb = _BB1

    def rows_spec(L):
        return pl.BlockSpec((bb * L, 2 * D), lambda i: (i, 0))

    def x_spec(L):
        return pl.BlockSpec((bb * L // 2, 2 * D), lambda i: (i, 0))

    def ridx_spec(L):
        return pl.BlockSpec((bb, L), lambda i: (i, 0))

    def w_specs(ws):
        return [pl.BlockSpec(w.shape, lambda i: (0, 0)) for w in ws]

    in_specs = (
        [rows_spec(L_I), ridx_spec(L_I), rows_spec(1),
         rows_spec(400), ridx_spec(400), rows_spec(L_S),
         rows_spec(L_IU), ridx_spec(L_IU), rows_spec(1)]
        + w_specs(wI) + w_specs(wO) + w_specs(wit))
    out_specs = [x_spec(L_I), ridx_spec(L_I),
                 x_spec(400), ridx_spec(400),
                 x_spec(L_IU), ridx_spec(L_IU)]
    out_shape = [
        jax.ShapeDtypeStruct((B * L_I // 2, 2 * D), BF16),
        jax.ShapeDtypeStruct((B, L_I), F32),
        jax.ShapeDtypeStruct((B * 400 // 2, 2 * D), BF16),
        jax.ShapeDtypeStruct((B, 400), F32),
        jax.ShapeDtypeStruct((B * L_IU // 2, 2 * D), BF16),
        jax.ShapeDtypeStruct((B, L_IU), F32),
    ]
    return pl.pallas_call(
        _pass1_body,
        grid=(nb,),
        in_specs=in_specs,
        out_specs=out_specs,
        out_shape=out_shape,
    )(qa, ria, pi, qao, ro, po, pt, rit, qj, *wI, *wO, *wit)


def _stats_body(lI, lO, lit, sI, sO, sit):
    for l_ref, s_ref in ((lI, sI), (lO, sO), (lit, sit)):
        l = l_ref[...]
        m = jnp.max(l, axis=0, keepdims=True)
        s = jnp.sum(jnp.exp(l - m), axis=0, keepdims=True)
        s_ref[...] = jnp.concatenate([m, s], axis=0)


def _stats(lI, lO, lit):
    return pl.pallas_call(
        _stats_body,
        out_shape=[jax.ShapeDtypeStruct((2, L_I), F32),
                   jax.ShapeDtypeStruct((2, 400), F32),
                   jax.ShapeDtypeStruct((2, L_IU), F32)],
    )(lI, lO, lit)


_BB2 = 32


def _agg(x_ref, l_ref, st_ref, W, b, L, M, bb):
    st = st_ref[...]
    a = jnp.exp(l_ref[...] - st[0:1]) / st[1:2]
    x = _unpack_pairs(x_ref[...].astype(F32), bb * L).reshape(bb, L, D)
    y = lax.broadcast_in_dim(a, (bb, L, D), (0, 1)) * x
    S = jnp.sum(y.reshape(bb, L // M, M, D), axis=1)
    return jnp.maximum(
        jnp.dot(S.reshape(bb * M, D), W, preferred_element_type=F32) + b, 0.0)


def _pass2_body(xia, lI, xoa, lO, fjt, lit, sI, sO, sit, pi,
                WI, bI, WO, bO, WL, bL, A1St, A1Sb, c1S, a2S, c2S,
                hiI_o, hoI_o, hb_o, zj_o):
    bb = _BB2
    hiI_o[...] = _agg(xia, lI, sI, WI[...], bI[...], L_I, 1, bb)
    hoI = _agg(xoa, lO, sO, WO[...], bO[...], 400, L_S, bb)
    hoI_o[...] = hoI
    zj_o[...] = _agg(fjt, lit, sit, WL[...], bL[...], L_IU, 1, bb)
    piraw = pi[...][:, D:]
    piA = jnp.dot(piraw, A1St[...], preferred_element_type=F32) + c1S[...]
    hob = jnp.dot(hoI, A1Sb[...], preferred_element_type=F32)
    h = jnp.maximum(hob.reshape(bb, L_S, D) + piA.reshape(bb, 1, D), 0.0)
    hb_o[...] = jnp.sum(h * a2S[...], axis=-1) + c2S[0, 0]


def _pass2(xia, lI, xoa, lO, fjt, lit, sI, sO, sit, pi, ws):
    bb = _BB2
    nb = B // bb

    def x_spec(L):
        return pl.BlockSpec((bb * L // 2, 2 * D), lambda i: (i, 0))

    def rows_spec(L):
        return pl.BlockSpec((bb * L, D), lambda i: (i, 0))

    def l_spec(L):
        return pl.BlockSpec((bb, L), lambda i: (i, 0))

    def full(a):
        return pl.BlockSpec(a.shape, lambda i: (0, 0))

    in_specs = ([x_spec(L_I), l_spec(L_I), x_spec(400), l_spec(400),
                 x_spec(L_IU), l_spec(L_IU)]
                + [full(s) for s in (sI, sO, sit)]
                + [pl.BlockSpec((bb, 2 * D), lambda i: (i, 0))]
                + [full(w) for w in ws])
    out_specs = [rows_spec(1), rows_spec(L_S), l_spec(L_S), rows_spec(1)]
    out_shape = [jax.ShapeDtypeStruct((B, D), F32),
                 jax.ShapeDtypeStruct((B * L_S, D), F32),
                 jax.ShapeDtypeStruct((B, L_S), F32),
                 jax.ShapeDtypeStruct((B, D), F32)]
    return pl.pallas_call(
        _pass2_body,
        grid=(nb,),
        in_specs=in_specs,
        out_specs=out_specs,
        out_shape=out_shape,
    )(xia, lI, xoa, lO, fjt, lit, sI, sO, sit, pi, *ws)


def _mlp3(x, W1, b1, W2, b2, W3, b3):
    h = jnp.maximum(jnp.dot(x, W1, preferred_element_type=F32) + b1, 0.0)
    h = jnp.maximum(jnp.dot(h, W2, preferred_element_type=F32) + b2, 0.0)
    return jnp.dot(h, W3, preferred_element_type=F32) + b3


def _pass3_body(hiI, hoI, hb, zj,
                WO, bO, W1u, b1u, W2u, b2u, W3u, b3u,
                w1g, b1g, w2g, b2g, w3g, b3g, out):
    h = hb[...]
    m = jnp.max(h, axis=0, keepdims=True)
    e = jnp.exp(h - m)
    beta = e / jnp.sum(e, axis=0, keepdims=True)
    ho = hoI[...].reshape(B, L_S, D)
    Sp = jnp.sum(lax.broadcast_in_dim(beta, (B, L_S, D), (0, 1)) * ho,
                 axis=1)
    hiS = jnp.maximum(
        jnp.dot(Sp, WO[...], preferred_element_type=F32) + bO[...], 0.0)
    xu = jnp.concatenate([hiI[...], hiS], axis=1)
    hi = _mlp3(xu, W1u[...], b1u[...], W2u[...], b2u[...], W3u[...], b3u[...])
    xg = jnp.concatenate([hi, zj[...]], axis=1)
    g1 = jnp.maximum(
        jnp.sum(xg * w1g[...], axis=1, keepdims=True) + b1g[0, 0], 0.0)
    g2 = jnp.maximum(g1 * w2g[0, 0] + b2g[0, 0], 0.0)
    out[...] = g2 * w3g[0, 0] + b3g[0, 0]


def _pass3(hiI, hoI, hb, zj, ws):
    return pl.pallas_call(
        _pass3_body,
        out_shape=jax.ShapeDtypeStruct((B, 1), F32),
    )(hiI, hoI, hb, zj, *ws)


def _row2(v):
    return v.reshape(1, -1)


def _prep_stream(embed_r, mlp, att):
    W1, b1, W2, b2, W3, b3 = mlp
    A1, c1, A2, c2 = att
    rtab = embed_r @ W1[D:] + b1
    rtab8 = jnp.concatenate([rtab, jnp.zeros((3, D), F32)], axis=0)
    Wf = jnp.concatenate([W3, W3 @ A1[D:]], axis=1)
    bf = jnp.concatenate([b3, b3 @ A1[D:]], axis=0)
    return (rtab8, W2, _row2(b2), Wf, _row2(bf),
            A1[:D], _row2(c1), _row2(A2[:, 0]), _row2(c2))


def kernel(params, nodes_u, nodes_i, u_items_list, u_users_list, i_users_list,
           u_items_r, u_users_items, u_users_items_r, i_users_r):
    p = params
    i32 = jnp.int32
    idx_u = jnp.concatenate([nodes_u.astype(i32),
                             u_users_list.astype(i32).reshape(-1),
                             i_users_list.astype(i32).reshape(-1)])
    idx_i = jnp.concatenate([nodes_i.astype(i32),
                             u_items_list.astype(i32).reshape(-1),
                             u_users_items.astype(i32).reshape(-1)])

    tbl_u, tbl_i = _prep_tables(p['embed_u'], p['embed_i'],
                                p['gu'][0][:D], p['gv'][0][:D])
    pi, po, pt, qj, qa, qao = _sc_gather(tbl_u, tbl_i, idx_u, idx_i)

    wI = _prep_stream(p['embed_r'], p['gv'], p['att_I'])
    wO = _prep_stream(p['embed_r'], p['gv'], p['att_O'])
    wit = _prep_stream(p['embed_r'], p['gu'], p['att_item'])

    ria = u_items_r.astype(i32)
    ro = u_users_items_r.astype(i32).reshape(B, 400)
    rit = i_users_r.astype(i32)

    xia, lI, xoa, lO, fjt, lit = _pass1(qa, ria, pi, qao, ro, po,
                                        pt, rit, qj, wI, wO, wit)
    sI, sO, sit = _stats(lI, lO, lit)

    WI, bI = p['linI']
    WO, bO = p['linO']
    WL, bL = p['lin_item']
    A1S, c1S, A2S, c2S = p['att_S']
    ws2 = [WI, _row2(bI), WO, _row2(bO), WL, _row2(bL),
           A1S[:D], A1S[D:], _row2(c1S), _row2(A2S[:, 0]),
           c2S.reshape(1, 1)]
    hiI, hoI, hb, zj = _pass2(xia, lI, xoa, lO, fjt, lit, sI, sO, sit, pi, ws2)

    W1u, b1u, W2u, b2u, W3u, b3u = p['umlp']
    w1g, b1g, w2g, b2g, w3g, b3g = p['g']
    ws3 = [WO, _row2(bO), W1u, _row2(b1u), W2u, _row2(b2u), W3u, _row2(b3u),
           _row2(w1g[:, 0]), b1g.reshape(1, 1), w2g.reshape(1, 1),
           b2g.reshape(1, 1), w3g.reshape(1, 1), b3g.reshape(1, 1)]
    return _pass3(hiI, hoI, hb, zj, ws3)
